# baseline (speedup 1.0000x reference)
import functools

import jax
import jax.numpy as jnp
from jax import lax
from jax.experimental import pallas as pl
from jax.experimental.pallas import tpu as pltpu
from jax.experimental.pallas import tpu_sc as plsc

_VOCAB = 45056
_EMB = 520
_B = 1024
_L = 50
_GCOLS = 32

_NC = 2
_NS = 16
_NW = _NC * _NS
_BPW = _B // _NW


def _make_fold_body(c0, sign):
    def body(embt_ref, w1_ref, avgw_ref, out_ref):
        w1h = w1_ref[0][:, c0:c0 + 512]
        mtb = jnp.concatenate(
            [w1h, jnp.zeros((16, 512), jnp.float32)], axis=0)
        mta = jnp.concatenate(
            [jnp.zeros((16, 8), jnp.float32),
             jnp.broadcast_to(sign * avgw_ref[...], (8, 8)),
             jnp.zeros((8, 8), jnp.float32)], axis=0)
        acc = lax.dot_general(
            embt_ref[8:_EMB, :], mtb, (((0,), (1,)), ((), ())),
            preferred_element_type=jnp.float32,
        )
        acc += lax.dot_general(
            embt_ref[0:8, :], mta, (((0,), (1,)), ((), ())),
            preferred_element_type=jnp.float32,
        )
        out_ref[:, 0:_GCOLS] = acc
    return body


def _fold_table(embt, fc1_w, avg_w, c0, sign, block_cols=5632):
    nb = _VOCAB // block_cols
    return pl.pallas_call(
        _make_fold_body(c0, sign),
        grid=(nb,),
        in_specs=[
            pl.BlockSpec((_EMB, block_cols), lambda i: (0, i)),
            pl.BlockSpec((1, 16, 1024), lambda i: (7, 0, 0)),
            pl.BlockSpec((1, 8), lambda i: (0, 0)),
        ],
        out_specs=pl.BlockSpec((block_cols, 128), lambda i: (i, 0)),
        out_shape=jax.ShapeDtypeStruct((_VOCAB, 128), jnp.float32),
    )(embt, fc1_w, avg_w)


def _bagsum_tile(g_hbm, idx_hbm, out_hbm, idx_v, rows_v, out_v, sem):
    wid = lax.axis_index("s") * _NC + lax.axis_index("c")
    base = wid * (_BPW // 2)
    pltpu.sync_copy(idx_hbm.at[pl.ds(base, _BPW // 2)], idx_v)

    copies = [
        pltpu.async_copy(g_hbm.at[idx_v.at[k]], rows_v.at[k], sem)
        for k in range(_BPW // 2)
    ]

    zero = jnp.zeros((16,), jnp.float32)
    for k in range(_BPW // 2):
        copies[k].wait()
        for h in range(2):
            def body(r, carry):
                a0, a1 = carry
                r = r + h * _L
                return a0 + rows_v[k, r, 0:16], a1 + rows_v[k, r, 16:32]

            a0, a1 = lax.fori_loop(0, _L, body, (zero, zero), unroll=5)
            out_v[2 * k + h, 0:16] = a0
            out_v[2 * k + h, 16:32] = a1

    pltpu.sync_copy(out_v, out_hbm.at[pl.ds(wid * _BPW, _BPW)])


def _bagsum(g128, idx4):
    g = g128.reshape(4 * _VOCAB, _GCOLS)
    mesh = plsc.VectorSubcoreMesh(core_axis_name="c", subcore_axis_name="s")
    kern = functools.partial(
        pl.kernel,
        out_type=jax.ShapeDtypeStruct((_B, 128), jnp.float32),
        mesh=mesh,
        scratch_types=[
            pltpu.VMEM((_BPW // 2, 2 * _L), jnp.int32),
            pltpu.VMEM((_BPW // 2, 2 * _L, _GCOLS), jnp.float32),
            pltpu.VMEM((_BPW, 128), jnp.float32),
            pltpu.SemaphoreType.DMA,
        ],
        compiler_params=pltpu.CompilerParams(use_tc_tiling_on_sc=False),
    )(_bagsum_tile)
    return kern(g, idx4.reshape(_B // 2, 2 * _L))


def _mlp_body(bso_ref, bsp_ref, w2_ref, w3_ref, b1_ref, b2_ref, b3_ref,
              avgb_ref, out_ref):
    bs = bso_ref[:, 0:_GCOLS] + bsp_ref[:, 0:_GCOLS]
    h1 = jnp.clip(bs[:, 0:16] + b1_ref[7:8, :], 0.0, 1.0)
    h2 = lax.dot_general(
        h1, w2_ref[0], (((1,), (1,)), ((), ())),
        preferred_element_type=jnp.float32,
    )
    h2 = jnp.clip(h2 + b2_ref[7:8, :], 0.0, 1.0)
    lane = lax.broadcasted_iota(jnp.int32, (_GCOLS, 128), 1)
    sub = lax.broadcasted_iota(jnp.int32, (_GCOLS, 128), 0)
    sel_sum = jnp.where(lane == 0, 1.0, 0.0)
    sel_avg = jnp.where((lane == 0) & (sub == 16), 1.0, 0.0)
    out = jnp.dot(h2 * w3_ref[0], sel_sum, preferred_element_type=jnp.float32)
    out += jnp.dot(bs, sel_avg, preferred_element_type=jnp.float32)
    out_ref[...] = out + (b3_ref[7, 0] + avgb_ref[0, 0])


def _mlp(bs_own, bs_opp, fc2_w, fc3_w, fc1_b, fc2_b, fc3_b, avg_b):
    return pl.pallas_call(
        _mlp_body,
        grid=(1,),
        in_specs=[
            pl.BlockSpec((_B, 128), lambda i: (0, 0)),
            pl.BlockSpec((_B, 128), lambda i: (0, 0)),
            pl.BlockSpec((1, 32, 16), lambda i: (7, 0, 0)),
            pl.BlockSpec((1, 1, 32), lambda i: (7, 0, 0)),
            pl.BlockSpec((8, 16), lambda i: (0, 0)),
            pl.BlockSpec((8, 32), lambda i: (0, 0)),
            pl.BlockSpec((8, 1), lambda i: (0, 0)),
            pl.BlockSpec((1, 1), lambda i: (0, 0)),
        ],
        out_specs=pl.BlockSpec((_B, 128), lambda i: (0, 0)),
        out_shape=jax.ShapeDtypeStruct((_B, 128), jnp.float32),
    )(bs_own, bs_opp, fc2_w, fc3_w, fc1_b, fc2_b, fc3_b, avg_b)


def kernel(own_batch, opp_batch, emb_own, emb_opp, avg_W, avg_b,
           fc1_W, fc1_b, fc2_W, fc2_b, fc3_W, fc3_b):
    idx_own = own_batch.astype(jnp.int32) * 4
    idx_opp = opp_batch.astype(jnp.int32) * 4

    g_own = _fold_table(emb_own.T, fc1_W, avg_W, 0, 1.0)
    bs_own = _bagsum(g_own, idx_own)
    g_opp = _fold_table(emb_opp.T, fc1_W, avg_W, 512, -1.0)
    bs_opp = _bagsum(g_opp, idx_opp)

    out = _mlp(bs_own, bs_opp, fc2_W, fc3_W, fc1_b, fc2_b,
               fc3_b, avg_b.reshape(1, 1))
    return out[:, 0]


# speedup vs baseline: 1.0173x; 1.0173x over previous
import functools

import jax
import jax.numpy as jnp
from jax import lax
from jax.experimental import pallas as pl
from jax.experimental.pallas import tpu as pltpu
from jax.experimental.pallas import tpu_sc as plsc

_VOCAB = 45056
_EMB = 520
_B = 1024
_L = 50
_GCOLS = 32

_NC = 2
_NS = 16
_NW = _NC * _NS
_BPW = _B // _NW


def _make_fold_body(c0, sign):
    def body(embt_ref, w1_ref, avgw_ref, out_ref):
        w1h = w1_ref[0][:, c0:c0 + 512]
        mtb = jnp.concatenate(
            [w1h, jnp.zeros((16, 512), jnp.float32)], axis=0)
        mta = jnp.concatenate(
            [jnp.zeros((16, 8), jnp.float32),
             jnp.broadcast_to(sign * avgw_ref[...], (8, 8)),
             jnp.zeros((8, 8), jnp.float32)], axis=0)
        acc = lax.dot_general(
            embt_ref[8:_EMB, :], mtb, (((0,), (1,)), ((), ())),
            preferred_element_type=jnp.float32,
        )
        acc += lax.dot_general(
            embt_ref[0:8, :], mta, (((0,), (1,)), ((), ())),
            preferred_element_type=jnp.float32,
        )
        out_ref[:, 0:_GCOLS] = acc
    return body


def _fold_table(embt, fc1_w, avg_w, c0, sign, block_cols=5632):
    nb = _VOCAB // block_cols
    return pl.pallas_call(
        _make_fold_body(c0, sign),
        grid=(nb,),
        in_specs=[
            pl.BlockSpec((_EMB, block_cols), lambda i: (0, i)),
            pl.BlockSpec((1, 16, 1024), lambda i: (7, 0, 0)),
            pl.BlockSpec((1, 8), lambda i: (0, 0)),
        ],
        out_specs=pl.BlockSpec((block_cols, 128), lambda i: (i, 0)),
        out_shape=jax.ShapeDtypeStruct((_VOCAB, 128), jnp.float32),
    )(embt, fc1_w, avg_w)


def _bagsum_tile(g_hbm, idx_hbm, out_hbm, idx_v, rows_v, out_v, sem):
    wid = lax.axis_index("s") * _NC + lax.axis_index("c")
    base = wid * _BPW
    pltpu.sync_copy(idx_hbm.at[pl.ds(base, _BPW)], idx_v)

    copies = [
        pltpu.async_copy(g_hbm.at[idx_v.at[b]], rows_v.at[b], sem)
        for b in range(_BPW)
    ]

    zero = jnp.zeros((16,), jnp.float32)
    for b in range(_BPW):
        copies[b].wait()

        def body(r, carry):
            a0, a1 = carry
            return a0 + rows_v[b, r, 0:16], a1 + rows_v[b, r, 16:32]

        a0, a1 = lax.fori_loop(0, _L, body, (zero, zero), unroll=5)
        out_v[b, 0:16] = a0
        out_v[b, 16:32] = a1

    pltpu.sync_copy(out_v, out_hbm.at[pl.ds(base, _BPW)])


def _bagsum(g128, idx4):
    g = g128.reshape(4 * _VOCAB, _GCOLS)
    mesh = plsc.VectorSubcoreMesh(core_axis_name="c", subcore_axis_name="s")
    kern = functools.partial(
        pl.kernel,
        out_type=jax.ShapeDtypeStruct((_B, 128), jnp.float32),
        mesh=mesh,
        scratch_types=[
            pltpu.VMEM((_BPW, _L), jnp.int32),
            pltpu.VMEM((_BPW, _L, _GCOLS), jnp.float32),
            pltpu.VMEM((_BPW, 128), jnp.float32),
            pltpu.SemaphoreType.DMA,
        ],
        compiler_params=pltpu.CompilerParams(use_tc_tiling_on_sc=False),
    )(_bagsum_tile)
    return kern(g, idx4)


def _mlp_body(bso_ref, bsp_ref, w2_ref, w3_ref, b1_ref, b2_ref, b3_ref,
              avgb_ref, out_ref):
    bs = bso_ref[:, 0:_GCOLS] + bsp_ref[:, 0:_GCOLS]
    h1 = jnp.clip(bs[:, 0:16] + b1_ref[7:8, :], 0.0, 1.0)
    h2 = lax.dot_general(
        h1, w2_ref[0], (((1,), (1,)), ((), ())),
        preferred_element_type=jnp.float32,
    )
    h2 = jnp.clip(h2 + b2_ref[7:8, :], 0.0, 1.0)
    lane = lax.broadcasted_iota(jnp.int32, (_GCOLS, 128), 1)
    sub = lax.broadcasted_iota(jnp.int32, (_GCOLS, 128), 0)
    sel_sum = jnp.where(lane == 0, 1.0, 0.0)
    sel_avg = jnp.where((lane == 0) & (sub == 16), 1.0, 0.0)
    out = jnp.dot(h2 * w3_ref[0], sel_sum, preferred_element_type=jnp.float32)
    out += jnp.dot(bs, sel_avg, preferred_element_type=jnp.float32)
    out_ref[...] = out + (b3_ref[7, 0] + avgb_ref[0, 0])


def _mlp(bs_own, bs_opp, fc2_w, fc3_w, fc1_b, fc2_b, fc3_b, avg_b):
    return pl.pallas_call(
        _mlp_body,
        grid=(1,),
        in_specs=[
            pl.BlockSpec((_B, 128), lambda i: (0, 0)),
            pl.BlockSpec((_B, 128), lambda i: (0, 0)),
            pl.BlockSpec((1, 32, 16), lambda i: (7, 0, 0)),
            pl.BlockSpec((1, 1, 32), lambda i: (7, 0, 0)),
            pl.BlockSpec((8, 16), lambda i: (0, 0)),
            pl.BlockSpec((8, 32), lambda i: (0, 0)),
            pl.BlockSpec((8, 1), lambda i: (0, 0)),
            pl.BlockSpec((1, 1), lambda i: (0, 0)),
        ],
        out_specs=pl.BlockSpec((_B, 128), lambda i: (0, 0)),
        out_shape=jax.ShapeDtypeStruct((_B, 128), jnp.float32),
    )(bs_own, bs_opp, fc2_w, fc3_w, fc1_b, fc2_b, fc3_b, avg_b)


def kernel(own_batch, opp_batch, emb_own, emb_opp, avg_W, avg_b,
           fc1_W, fc1_b, fc2_W, fc2_b, fc3_W, fc3_b):
    idx_own = own_batch.astype(jnp.int32) * 4
    idx_opp = opp_batch.astype(jnp.int32) * 4

    g_own = _fold_table(emb_own.T, fc1_W, avg_W, 0, 1.0)
    bs_own = _bagsum(g_own, idx_own)
    g_opp = _fold_table(emb_opp.T, fc1_W, avg_W, 512, -1.0)
    bs_opp = _bagsum(g_opp, idx_opp)

    out = _mlp(bs_own, bs_opp, fc2_W, fc3_W, fc1_b, fc2_b,
               fc3_b, avg_b.reshape(1, 1))
    return out[:, 0]
